# initial kernel scaffold (unmeasured)
import jax
import jax.numpy as jnp
from jax import lax
from jax.experimental import pallas as pl
from jax.experimental.pallas import tpu as pltpu


def kernel(
    x,
):
    def body(*refs):
        pass

    out_shape = jax.ShapeDtypeStruct(..., jnp.float32)
    return pl.pallas_call(body, out_shape=out_shape)(...)


# baseline (device time: 23195 ns/iter reference)
import jax
import jax.numpy as jnp
from jax import lax
from jax.experimental import pallas as pl
from jax.experimental.pallas import tpu as pltpu

N_DEV = 8


def kernel(x):
    m, n = x.shape
    c = m // N_DEV

    def body(x_ref, out_ref, gbuf, red_ref, p1_send, p1_recv, p2_send, p2_recv):
        me = lax.axis_index("i")

        barrier = pltpu.get_barrier_semaphore()
        for off in range(1, N_DEV):
            peer = (me + off) % N_DEV
            pl.semaphore_signal(
                barrier, inc=1,
                device_id=(peer,), device_id_type=pl.DeviceIdType.MESH,
            )
        pl.semaphore_wait(barrier, N_DEV - 1)

        p1_rdmas = []
        for off in range(1, N_DEV):
            peer = (me + off) % N_DEV
            rdma = pltpu.make_async_remote_copy(
                src_ref=x_ref.at[pl.ds(peer * c, c), :],
                dst_ref=gbuf.at[me],
                send_sem=p1_send.at[off],
                recv_sem=p1_recv.at[me],
                device_id=(peer,),
                device_id_type=pl.DeviceIdType.MESH,
            )
            rdma.start()
            p1_rdmas.append(rdma)

        acc = x_ref[pl.ds(me * c, c), :]
        for off in range(1, N_DEV):
            src = (me + off) % N_DEV
            recv = pltpu.make_async_remote_copy(
                src_ref=gbuf.at[src],
                dst_ref=gbuf.at[src],
                send_sem=p1_send.at[off],
                recv_sem=p1_recv.at[src],
                device_id=(src,),
                device_id_type=pl.DeviceIdType.MESH,
            )
            recv.wait_recv()
            acc = acc + gbuf[src]
        red_ref[:, :] = acc
        out_ref[pl.ds(me * c, c), :] = acc

        p2_rdmas = []
        for off in range(1, N_DEV):
            peer = (me + off) % N_DEV
            rdma = pltpu.make_async_remote_copy(
                src_ref=red_ref,
                dst_ref=out_ref.at[pl.ds(me * c, c), :],
                send_sem=p2_send.at[off],
                recv_sem=p2_recv.at[me],
                device_id=(peer,),
                device_id_type=pl.DeviceIdType.MESH,
            )
            rdma.start()
            p2_rdmas.append(rdma)

        for off in range(1, N_DEV):
            src = (me + off) % N_DEV
            recv = pltpu.make_async_remote_copy(
                src_ref=red_ref,
                dst_ref=out_ref.at[pl.ds(src * c, c), :],
                send_sem=p2_send.at[off],
                recv_sem=p2_recv.at[src],
                device_id=(src,),
                device_id_type=pl.DeviceIdType.MESH,
            )
            recv.wait_recv()

        for rdma in p1_rdmas:
            rdma.wait_send()
        for rdma in p2_rdmas:
            rdma.wait_send()

    return pl.pallas_call(
        body,
        out_shape=jax.ShapeDtypeStruct((m, n), x.dtype),
        in_specs=[pl.BlockSpec(memory_space=pltpu.VMEM)],
        out_specs=pl.BlockSpec(memory_space=pltpu.VMEM),
        scratch_shapes=[
            pltpu.VMEM((N_DEV, c, n), x.dtype),
            pltpu.VMEM((c, n), x.dtype),
            pltpu.SemaphoreType.DMA((N_DEV,)),
            pltpu.SemaphoreType.DMA((N_DEV,)),
            pltpu.SemaphoreType.DMA((N_DEV,)),
            pltpu.SemaphoreType.DMA((N_DEV,)),
        ],
        compiler_params=pltpu.CompilerParams(collective_id=0),
    )(x)


# device time: 20890 ns/iter; 1.1103x vs baseline; 1.1103x over previous
import jax
import jax.numpy as jnp
from jax import lax
from jax.experimental import pallas as pl
from jax.experimental.pallas import tpu as pltpu

N_DEV = 8
H = 4


def kernel(x):
    m, n = x.shape
    c = m // N_DEV
    w = n // H

    def body(x_ref, out_ref, gbuf, p1_send, p1_recv, p2_send, p2_recv):
        me = lax.axis_index("i")

        barrier = pltpu.get_barrier_semaphore()
        for off in range(1, N_DEV):
            peer = (me + off) % N_DEV
            pl.semaphore_signal(
                barrier, inc=1,
                device_id=(peer,), device_id_type=pl.DeviceIdType.MESH,
            )
        pl.semaphore_wait(barrier, N_DEV - 1)

        p1_rdmas = []
        for h in range(H):
            for off in range(1, N_DEV):
                peer = (me + off) % N_DEV
                rdma = pltpu.make_async_remote_copy(
                    src_ref=x_ref.at[pl.ds(peer * c, c), pl.ds(h * w, w)],
                    dst_ref=gbuf.at[me, :, pl.ds(h * w, w)],
                    send_sem=p1_send.at[h, off],
                    recv_sem=p1_recv.at[h, me],
                    device_id=(peer,),
                    device_id_type=pl.DeviceIdType.MESH,
                )
                rdma.start()
                p1_rdmas.append(rdma)

        p2_rdmas = []
        for h in range(H):
            acc = x_ref[pl.ds(me * c, c), pl.ds(h * w, w)]
            for off in range(1, N_DEV):
                src = (me + off) % N_DEV
                recv = pltpu.make_async_remote_copy(
                    src_ref=gbuf.at[src, :, pl.ds(h * w, w)],
                    dst_ref=gbuf.at[src, :, pl.ds(h * w, w)],
                    send_sem=p1_send.at[h, off],
                    recv_sem=p1_recv.at[h, src],
                    device_id=(src,),
                    device_id_type=pl.DeviceIdType.MESH,
                )
                recv.wait_recv()
                acc = acc + gbuf[src, :, pl.ds(h * w, w)]
            out_ref[pl.ds(me * c, c), pl.ds(h * w, w)] = acc

            for off in range(1, N_DEV):
                peer = (me + off) % N_DEV
                rdma = pltpu.make_async_remote_copy(
                    src_ref=out_ref.at[pl.ds(me * c, c), pl.ds(h * w, w)],
                    dst_ref=out_ref.at[pl.ds(me * c, c), pl.ds(h * w, w)],
                    send_sem=p2_send.at[h, off],
                    recv_sem=p2_recv.at[h, me],
                    device_id=(peer,),
                    device_id_type=pl.DeviceIdType.MESH,
                )
                rdma.start()
                p2_rdmas.append(rdma)

        for h in range(H):
            for off in range(1, N_DEV):
                src = (me + off) % N_DEV
                recv = pltpu.make_async_remote_copy(
                    src_ref=out_ref.at[pl.ds(src * c, c), pl.ds(h * w, w)],
                    dst_ref=out_ref.at[pl.ds(src * c, c), pl.ds(h * w, w)],
                    send_sem=p2_send.at[h, off],
                    recv_sem=p2_recv.at[h, src],
                    device_id=(src,),
                    device_id_type=pl.DeviceIdType.MESH,
                )
                recv.wait_recv()

        for rdma in p1_rdmas:
            rdma.wait_send()
        for rdma in p2_rdmas:
            rdma.wait_send()

    return pl.pallas_call(
        body,
        out_shape=jax.ShapeDtypeStruct((m, n), x.dtype),
        in_specs=[pl.BlockSpec(memory_space=pltpu.VMEM)],
        out_specs=pl.BlockSpec(memory_space=pltpu.VMEM),
        scratch_shapes=[
            pltpu.VMEM((N_DEV, c, n), x.dtype),
            pltpu.SemaphoreType.DMA((H, N_DEV)),
            pltpu.SemaphoreType.DMA((H, N_DEV)),
            pltpu.SemaphoreType.DMA((H, N_DEV)),
            pltpu.SemaphoreType.DMA((H, N_DEV)),
        ],
        compiler_params=pltpu.CompilerParams(collective_id=0),
    )(x)
